# baseline (speedup 1.0000x reference)
import functools

import jax
import jax.numpy as jnp
from jax import lax
from jax.experimental import pallas as pl
from jax.experimental.pallas import tpu as pltpu
from jax.experimental.pallas import tpu_sc as plsc

_C = 80
_B = 2
_R = _B * _C
_N = 16128
_NP = 16384
_K = 1000
_KP = 1024
_M = 200
_TH = 0.6
_BIG = 10 ** 9
_NW = 32
_RPW = _R // _NW


def _decode_body(cls_ref, obj_ref, bx_ref, by_ref, bw_ref, bh_ref,
                 sc_out, t_out, y0_out, x0_out, y1_out, x1_out):
    iota_n = jax.lax.broadcasted_iota(jnp.int32, (_R, _NP), 1)
    obj = jax.nn.sigmoid(obj_ref[...])
    cls = jax.nn.sigmoid(cls_ref[...])
    objr = jnp.concatenate(
        [jnp.broadcast_to(obj[b:b + 1, :], (_C, _NP)) for b in range(_B)],
        axis=0)
    scores = jnp.where(iota_n < _N, cls * objr, 0.0)
    sc_out[...] = scores

    bx, by = bx_ref[...], by_ref[...]
    bw, bh = bw_ref[...], bh_ref[...]
    y0_out[...] = by - bh * 0.5
    x0_out[...] = bx - bw * 0.5
    y1_out[...] = by + bh * 0.5
    x1_out[...] = bx + bw * 0.5

    sbits = jax.lax.bitcast_convert_type(scores, jnp.int32)

    def bs_body(i, t):
        bit = jnp.int32(30) - i
        cand = t | (jnp.left_shift(jnp.int32(1), bit))
        cnt = jnp.sum((sbits >= cand).astype(jnp.int32), axis=1, keepdims=True)
        return jnp.where(cnt >= _K, cand, t)

    t = jax.lax.fori_loop(0, 31, bs_body, jnp.zeros((_R, 1), jnp.int32))
    tf = jax.lax.bitcast_convert_type(t, jnp.float32)
    t_out[...] = jnp.broadcast_to(tf, (_R, 16))


def _decode(cls_t, obj, bx, by, bw, bh):
    f32, i32 = jnp.float32, jnp.int32
    outs = [
        jax.ShapeDtypeStruct((_R, _NP), f32),
        jax.ShapeDtypeStruct((_R, 16), f32),
        jax.ShapeDtypeStruct((_B, _NP), f32),
        jax.ShapeDtypeStruct((_B, _NP), f32),
        jax.ShapeDtypeStruct((_B, _NP), f32),
        jax.ShapeDtypeStruct((_B, _NP), f32),
    ]
    return pl.pallas_call(
        _decode_body,
        out_shape=outs,
        compiler_params=pltpu.CompilerParams(
            vmem_limit_bytes=110 * 1024 * 1024),
    )(cls_t, obj, bx, by, bw, bh)


def _compact_body(sc_hbm, t_hbm, y0_hbm, x0_hbm, y1_hbm, x1_hbm,
                  osc_hbm, oy0_hbm, ox0_hbm, oy1_hbm, ox1_hbm,
                  sbuf, tbuf, y0t, x0t, y1t, x1t,
                  scc, ixc, g0, g1, g2, g3, cntb, baseb):
    wid = lax.axis_index("c") * 16 + lax.axis_index("s")
    row0 = wid * _RPW
    b = row0 // _C
    pltpu.sync_copy(y0_hbm.at[b], y0t)
    pltpu.sync_copy(x0_hbm.at[b], x0t)
    pltpu.sync_copy(y1_hbm.at[b], y1t)
    pltpu.sync_copy(x1_hbm.at[b], x1t)
    lanes = lax.iota(jnp.int32, 16)
    lane0 = lanes == 0
    zf = jnp.zeros((16,), jnp.float32)
    zi = jnp.zeros((16,), jnp.int32)

    for r in range(_RPW):
        row = row0 + r
        pltpu.sync_copy(sc_hbm.at[row], sbuf)
        pltpu.sync_copy(t_hbm.at[row], tbuf)
        tv = tbuf[...]

        def zbody(j, _):
            scc[pl.ds(j * 16, 16)] = zf
            ixc[pl.ds(j * 16, 16)] = zi
            return 0

        lax.fori_loop(0, _KP // 16, zbody, 0)

        def c1body(i, _):
            v = sbuf[pl.ds(i * 16, 16)]
            m = v >= tv
            c = plsc.all_reduce_population_count(m)
            plsc.store_scatter(cntb, [jnp.zeros((16,), jnp.int32) + i], c,
                               mask=lane0)
            return 0

        lax.fori_loop(0, _NP // 16, c1body, 0)

        def c2body(i, cur):
            c16 = cntb[pl.ds(i * 16, 16)]
            pos = plsc.cumsum(c16)
            baseb[pl.ds(i * 16, 16)] = (cur + pos) - c16
            return cur + jnp.sum(c16)

        lax.fori_loop(0, _NP // 256, c2body, jnp.int32(0))

        def c3body(i, _):
            v = sbuf[pl.ds(i * 16, 16)]
            m = v >= tv
            mi = jnp.where(m, 1, 0)
            pos = plsc.cumsum(mi)
            base = plsc.load_gather(baseb, [jnp.zeros((16,), jnp.int32) + i])
            idxv = (base + pos) - 1
            okm = m & (idxv < _KP)
            plsc.store_scatter(scc, [idxv], v, mask=okm)
            plsc.store_scatter(ixc, [idxv], (i * 16) + lanes, mask=okm)
            return 0

        lax.fori_loop(0, _NP // 16, c3body, 0)

        def gbody(i, _):
            sl = pl.ds(i * 16, 16)
            ix = ixc[sl]
            g0[sl] = plsc.load_gather(y0t, [ix])
            g1[sl] = plsc.load_gather(x0t, [ix])
            g2[sl] = plsc.load_gather(y1t, [ix])
            g3[sl] = plsc.load_gather(x1t, [ix])
            return 0

        lax.fori_loop(0, _KP // 16, gbody, 0)

        pltpu.sync_copy(scc, osc_hbm.at[row])
        pltpu.sync_copy(g0, oy0_hbm.at[row])
        pltpu.sync_copy(g1, ox0_hbm.at[row])
        pltpu.sync_copy(g2, oy1_hbm.at[row])
        pltpu.sync_copy(g3, ox1_hbm.at[row])


def _compact(scores, t16, y0, x0, y1, x1):
    f32 = jnp.float32
    mesh = plsc.VectorSubcoreMesh(core_axis_name="c", subcore_axis_name="s")
    out_type = [jax.ShapeDtypeStruct((_R, _KP), f32) for _ in range(5)]
    scratch = (
        [pltpu.VMEM((_NP,), f32), pltpu.VMEM((16,), f32)]
        + [pltpu.VMEM((_NP,), f32) for _ in range(4)]
        + [pltpu.VMEM((_KP,), f32), pltpu.VMEM((_KP,), jnp.int32)]
        + [pltpu.VMEM((_KP,), f32) for _ in range(4)]
        + [pltpu.VMEM((_NP // 16,), jnp.int32) for _ in range(2)]
    )
    fn = functools.partial(
        pl.kernel, mesh=mesh, out_type=out_type, scratch_types=scratch,
        compiler_params=pltpu.CompilerParams(needs_layout_passes=False),
    )(_compact_body)
    return fn(scores, t16, y0, x0, y1, x1)


def _nms_body(sc_ref, y0_ref, x0_ref, y1_ref, x1_ref,
              s_surv, y0s, x0s, y1s, x1s,
              vscr, a2c):
    vscr[...] = sc_ref[...]
    y0a, x0a = y0_ref[...], x0_ref[...]
    y1a, x1a = y1_ref[...], x1_ref[...]
    a2c[...] = (jnp.maximum(y1a - y0a, 0.0) * jnp.maximum(x1a - x0a, 0.0))

    zs = jnp.zeros((_R, _M), jnp.float32)
    s_surv[...] = zs
    y0s[...] = zs
    x0s[...] = zs
    y1s[...] = zs
    x1s[...] = zs
    iob = jax.lax.broadcasted_iota(jnp.int32, (_R, _KP), 1)
    k_iota_m = jax.lax.broadcasted_iota(jnp.int32, (_R, _M), 1)

    def nms_round(k, _):
        slot = k_iota_m == k
        vb = vscr[...]
        m = jnp.max(vb, axis=1, keepdims=True)
        ii = jnp.min(jnp.where(vb == m, iob, _KP), axis=1, keepdims=True)
        onehot = iob == ii
        y0r, x0r = y0_ref[...], x0_ref[...]
        y1r, x1r = y1_ref[...], x1_ref[...]
        sy0 = jnp.sum(jnp.where(onehot, y0r, 0.0), axis=1, keepdims=True)
        sx0 = jnp.sum(jnp.where(onehot, x0r, 0.0), axis=1, keepdims=True)
        sy1 = jnp.sum(jnp.where(onehot, y1r, 0.0), axis=1, keepdims=True)
        sx1 = jnp.sum(jnp.where(onehot, x1r, 0.0), axis=1, keepdims=True)
        a1 = jnp.maximum(sy1 - sy0, 0.0) * jnp.maximum(sx1 - sx0, 0.0)
        inter = (jnp.maximum(jnp.minimum(sy1, y1r) - jnp.maximum(sy0, y0r), 0.0)
                 * jnp.maximum(jnp.minimum(sx1, x1r) - jnp.maximum(sx0, x0r), 0.0))
        union = a1 + a2c[...] - inter
        supp = (inter > _TH * union) & (union > 0.0)
        vscr[...] = jnp.where(supp | onehot, 0.0, vb)
        keep = m > 0.0

        def put(ref, val):
            v = jnp.where(keep, val, 0.0)
            ref[...] = ref[...] + jnp.where(
                slot, jnp.broadcast_to(v, (_R, _M)), 0.0)

        put(s_surv, m)
        put(y0s, sy0)
        put(x0s, sx0)
        put(y1s, sy1)
        put(x1s, sx1)
        return 0

    jax.lax.fori_loop(0, _M, nms_round, 0)


def _nms(sc_c, y0c, x0c, y1c, x1c):
    f32 = jnp.float32
    outs = [jax.ShapeDtypeStruct((_R, _M), f32) for _ in range(5)]
    scr = [pltpu.VMEM((_R, _KP), f32), pltpu.VMEM((_R, _KP), f32)]
    return pl.pallas_call(
        _nms_body,
        out_shape=outs,
        scratch_shapes=scr,
        compiler_params=pltpu.CompilerParams(
            vmem_limit_bytes=100 * 1024 * 1024),
    )(sc_c, y0c, x0c, y1c, x1c)


_NF = _C * _M
_NFP = 16384


def _sort_body(s_ref, y0_ref, x0_ref, y1_ref, x1_ref,
               sc_o, cl_o, y0_o, x0_o, y1_o, x1_o):
    ii = jax.lax.broadcasted_iota(jnp.int32, (_B, _NFP), 1)
    s = s_ref[...]
    idx = ii
    y0, x0 = y0_ref[...], x0_ref[...]
    y1, x1 = y1_ref[...], x1_ref[...]

    def pair(x, j, bit0):
        lo = pltpu.roll(x, (-j) % _NFP, 1)
        hi = pltpu.roll(x, j, 1)
        return jnp.where(bit0, lo, hi)

    k = 2
    while k <= _NFP:
        j = k // 2
        while j >= 1:
            bit0 = (ii & j) == 0
            keep_max = ((ii & k) == 0) == bit0
            ps = pair(s, j, bit0)
            pidx = pair(idx, j, bit0)
            py0 = pair(y0, j, bit0)
            px0 = pair(x0, j, bit0)
            py1 = pair(y1, j, bit0)
            px1 = pair(x1, j, bit0)
            greater = (s > ps) | ((s == ps) & (idx < pidx))
            take_self = keep_max == greater
            s = jnp.where(take_self, s, ps)
            idx = jnp.where(take_self, idx, pidx)
            y0 = jnp.where(take_self, y0, py0)
            x0 = jnp.where(take_self, x0, px0)
            y1 = jnp.where(take_self, y1, py1)
            x1 = jnp.where(take_self, x1, px1)
            j //= 2
        k *= 2

    sc_o[...] = s[:, :_M]
    cl_o[...] = (idx[:, :_M] // _M).astype(jnp.float32)
    y0_o[...] = y0[:, :_M]
    x0_o[...] = x0[:, :_M]
    y1_o[...] = y1[:, :_M]
    x1_o[...] = x1[:, :_M]


def _sort(sf, y0f, x0f, y1f, x1f):
    f32 = jnp.float32
    outs = [jax.ShapeDtypeStruct((_B, _M), f32) for _ in range(6)]
    return pl.pallas_call(
        _sort_body,
        out_shape=outs,
        compiler_params=pltpu.CompilerParams(
            vmem_limit_bytes=100 * 1024 * 1024),
    )(sf, y0f, x0f, y1f, x1f)


@jax.jit
def kernel(level_3, level_4, level_5):
    parts = []
    for x in (level_3, level_4, level_5):
        _, H, W, _ = x.shape
        parts.append(x.reshape(_B, H * W * 3, 85))
    d = jnp.concatenate(parts, axis=1)
    pad = _NP - _N
    bx = jnp.pad(d[..., 0], ((0, 0), (0, pad)))
    by = jnp.pad(d[..., 1], ((0, 0), (0, pad)))
    bw = jnp.pad(d[..., 2], ((0, 0), (0, pad)))
    bh = jnp.pad(d[..., 3], ((0, 0), (0, pad)))
    obj = jnp.pad(d[..., 4], ((0, 0), (0, pad)))
    cls_t = jnp.pad(jnp.transpose(d[..., 5:], (0, 2, 1)).reshape(_R, _N),
                    ((0, 0), (0, pad)))

    scores, t16, y0, x0, y1, x1 = _decode(cls_t, obj, bx, by, bw, bh)
    sc_c, y0c, x0c, y1c, x1c = _compact(scores, t16, y0, x0, y1, x1)
    ssv, y0s, x0s, y1s, x1s = _nms(sc_c, y0c, x0c, y1c, x1c)

    fpad = _NFP - _NF
    flat = lambda a, v: jnp.pad(a.reshape(_B, _NF), ((0, 0), (0, fpad)),
                                constant_values=v)
    sc, cl, fy0, fx0, fy1, fx1 = _sort(
        flat(ssv, -1.0), flat(y0s, 0.0), flat(x0s, 0.0),
        flat(y1s, 0.0), flat(x1s, 0.0))

    boxes = jnp.stack([fy0, fx0, fy1, fx1], axis=-1)
    return boxes, sc, cl


# speedup vs baseline: 1.0699x; 1.0699x over previous
import functools

import jax
import jax.numpy as jnp
from jax import lax
from jax.experimental import pallas as pl
from jax.experimental.pallas import tpu as pltpu
from jax.experimental.pallas import tpu_sc as plsc

_C = 80
_B = 2
_R = _B * _C
_N = 16128
_NP = 16384
_K = 1000
_KP = 1024
_M = 200
_TH = 0.6
_BIG = 10 ** 9
_NW = 32
_RPW = _R // _NW


def _decode_body(cls_ref, obj_ref, bx_ref, by_ref, bw_ref, bh_ref,
                 sc_out, t_out, y0_out, x0_out, y1_out, x1_out):
    iota_n = jax.lax.broadcasted_iota(jnp.int32, (_R, _NP), 1)
    obj = jax.nn.sigmoid(obj_ref[...])
    cls = jax.nn.sigmoid(cls_ref[...])
    objr = jnp.concatenate(
        [jnp.broadcast_to(obj[b:b + 1, :], (_C, _NP)) for b in range(_B)],
        axis=0)
    scores = jnp.where(iota_n < _N, cls * objr, 0.0)
    sc_out[...] = scores

    bx, by = bx_ref[...], by_ref[...]
    bw, bh = bw_ref[...], bh_ref[...]
    y0_out[...] = by - bh * 0.5
    x0_out[...] = bx - bw * 0.5
    y1_out[...] = by + bh * 0.5
    x1_out[...] = bx + bw * 0.5

    sbits = jax.lax.bitcast_convert_type(scores, jnp.int32)

    def bs_body(i, t):
        bit = jnp.int32(30) - i
        cand = t | (jnp.left_shift(jnp.int32(1), bit))
        cnt = jnp.sum((sbits >= cand).astype(jnp.int32), axis=1, keepdims=True)
        return jnp.where(cnt >= _K, cand, t)

    t = jax.lax.fori_loop(0, 31, bs_body, jnp.zeros((_R, 1), jnp.int32))
    tf = jax.lax.bitcast_convert_type(t, jnp.float32)
    t_out[...] = jnp.broadcast_to(tf, (_R, 16))


def _decode(cls_t, obj, bx, by, bw, bh):
    f32, i32 = jnp.float32, jnp.int32
    outs = [
        jax.ShapeDtypeStruct((_R, _NP), f32),
        jax.ShapeDtypeStruct((_R, 16), f32),
        jax.ShapeDtypeStruct((_B, _NP), f32),
        jax.ShapeDtypeStruct((_B, _NP), f32),
        jax.ShapeDtypeStruct((_B, _NP), f32),
        jax.ShapeDtypeStruct((_B, _NP), f32),
    ]
    return pl.pallas_call(
        _decode_body,
        out_shape=outs,
        compiler_params=pltpu.CompilerParams(
            vmem_limit_bytes=110 * 1024 * 1024),
    )(cls_t, obj, bx, by, bw, bh)


def _compact_body(sc_hbm, t_hbm, y0_hbm, x0_hbm, y1_hbm, x1_hbm,
                  osc_hbm, oy0_hbm, ox0_hbm, oy1_hbm, ox1_hbm,
                  sbuf, tbuf, y0t, x0t, y1t, x1t,
                  scc, ixc, g0, g1, g2, g3):
    wid = lax.axis_index("c") * 16 + lax.axis_index("s")
    row0 = wid * _RPW
    b = row0 // _C
    pltpu.sync_copy(y0_hbm.at[b], y0t)
    pltpu.sync_copy(x0_hbm.at[b], x0t)
    pltpu.sync_copy(y1_hbm.at[b], y1t)
    pltpu.sync_copy(x1_hbm.at[b], x1t)
    lanes = lax.iota(jnp.int32, 16)
    lane0 = lanes == 0
    zf = jnp.zeros((16,), jnp.float32)
    zi = jnp.zeros((16,), jnp.int32)

    for r in range(_RPW):
        row = row0 + r
        pltpu.sync_copy(sc_hbm.at[row], sbuf)
        pltpu.sync_copy(t_hbm.at[row], tbuf)
        tv = tbuf[...]

        def zbody(j, _):
            scc[pl.ds(j * 16, 16)] = zf
            ixc[pl.ds(j * 16, 16)] = zi
            return 0

        lax.fori_loop(0, _KP // 16, zbody, 0)

        def cbody(i, cur):
            v = sbuf[pl.ds(i * 16, 16)]
            m = v >= tv
            mi = jnp.where(m, 1, 0)
            pos = plsc.cumsum(mi)
            idxv = (cur + pos) - 1
            okm = m & (idxv < _KP)
            plsc.store_scatter(scc, [idxv], v, mask=okm)
            plsc.store_scatter(ixc, [idxv], (i * 16) + lanes, mask=okm)
            return cur + jnp.sum(mi)

        lax.fori_loop(0, _NP // 16, cbody, jnp.int32(0))

        def gbody(i, _):
            sl = pl.ds(i * 16, 16)
            ix = ixc[sl]
            g0[sl] = plsc.load_gather(y0t, [ix])
            g1[sl] = plsc.load_gather(x0t, [ix])
            g2[sl] = plsc.load_gather(y1t, [ix])
            g3[sl] = plsc.load_gather(x1t, [ix])
            return 0

        lax.fori_loop(0, _KP // 16, gbody, 0)

        pltpu.sync_copy(scc, osc_hbm.at[row])
        pltpu.sync_copy(g0, oy0_hbm.at[row])
        pltpu.sync_copy(g1, ox0_hbm.at[row])
        pltpu.sync_copy(g2, oy1_hbm.at[row])
        pltpu.sync_copy(g3, ox1_hbm.at[row])


def _compact(scores, t16, y0, x0, y1, x1):
    f32 = jnp.float32
    mesh = plsc.VectorSubcoreMesh(core_axis_name="c", subcore_axis_name="s")
    out_type = [jax.ShapeDtypeStruct((_R, _KP), f32) for _ in range(5)]
    scratch = (
        [pltpu.VMEM((_NP,), f32), pltpu.VMEM((16,), f32)]
        + [pltpu.VMEM((_NP,), f32) for _ in range(4)]
        + [pltpu.VMEM((_KP,), f32), pltpu.VMEM((_KP,), jnp.int32)]
        + [pltpu.VMEM((_KP,), f32) for _ in range(4)]
    )
    fn = functools.partial(
        pl.kernel, mesh=mesh, out_type=out_type, scratch_types=scratch,
        compiler_params=pltpu.CompilerParams(needs_layout_passes=False),
    )(_compact_body)
    return fn(scores, t16, y0, x0, y1, x1)


def _nms_body(sc_ref, y0_ref, x0_ref, y1_ref, x1_ref,
              s_surv, y0s, x0s, y1s, x1s,
              vscr, a2c):
    vscr[...] = sc_ref[...]
    y0a, x0a = y0_ref[...], x0_ref[...]
    y1a, x1a = y1_ref[...], x1_ref[...]
    a2c[...] = (jnp.maximum(y1a - y0a, 0.0) * jnp.maximum(x1a - x0a, 0.0))

    zs = jnp.zeros((_R, _M), jnp.float32)
    s_surv[...] = zs
    y0s[...] = zs
    x0s[...] = zs
    y1s[...] = zs
    x1s[...] = zs
    iob = jax.lax.broadcasted_iota(jnp.int32, (_R, _KP), 1)
    k_iota_m = jax.lax.broadcasted_iota(jnp.int32, (_R, _M), 1)

    def nms_round(k, _):
        slot = k_iota_m == k
        vb = vscr[...]
        m = jnp.max(vb, axis=1, keepdims=True)
        ii = jnp.min(jnp.where(vb == m, iob, _KP), axis=1, keepdims=True)
        onehot = iob == ii
        y0r, x0r = y0_ref[...], x0_ref[...]
        y1r, x1r = y1_ref[...], x1_ref[...]
        sy0 = jnp.sum(jnp.where(onehot, y0r, 0.0), axis=1, keepdims=True)
        sx0 = jnp.sum(jnp.where(onehot, x0r, 0.0), axis=1, keepdims=True)
        sy1 = jnp.sum(jnp.where(onehot, y1r, 0.0), axis=1, keepdims=True)
        sx1 = jnp.sum(jnp.where(onehot, x1r, 0.0), axis=1, keepdims=True)
        a1 = jnp.maximum(sy1 - sy0, 0.0) * jnp.maximum(sx1 - sx0, 0.0)
        inter = (jnp.maximum(jnp.minimum(sy1, y1r) - jnp.maximum(sy0, y0r), 0.0)
                 * jnp.maximum(jnp.minimum(sx1, x1r) - jnp.maximum(sx0, x0r), 0.0))
        union = a1 + a2c[...] - inter
        supp = inter > _TH * union
        vscr[...] = jnp.where(supp | onehot, 0.0, vb)
        keep = m > 0.0

        def put(ref, val, masked=True):
            v = jnp.where(keep, val, 0.0) if masked else val
            ref[...] = ref[...] + jnp.where(
                slot, jnp.broadcast_to(v, (_R, _M)), 0.0)

        put(s_surv, m, masked=False)
        put(y0s, sy0)
        put(x0s, sx0)
        put(y1s, sy1)
        put(x1s, sx1)
        return 0

    jax.lax.fori_loop(0, _M, nms_round, 0)


def _nms(sc_c, y0c, x0c, y1c, x1c):
    f32 = jnp.float32
    outs = [jax.ShapeDtypeStruct((_R, _M), f32) for _ in range(5)]
    scr = [pltpu.VMEM((_R, _KP), f32), pltpu.VMEM((_R, _KP), f32)]
    return pl.pallas_call(
        _nms_body,
        out_shape=outs,
        scratch_shapes=scr,
        compiler_params=pltpu.CompilerParams(
            vmem_limit_bytes=100 * 1024 * 1024),
    )(sc_c, y0c, x0c, y1c, x1c)


_NF = _C * _M
_NFP = 16384


def _sort_body(s_ref, y0_ref, x0_ref, y1_ref, x1_ref,
               sc_o, cl_o, y0_o, x0_o, y1_o, x1_o):
    ii = jax.lax.broadcasted_iota(jnp.int32, (_B, _NFP), 1)
    s = s_ref[...]
    idx = ii
    y0, x0 = y0_ref[...], x0_ref[...]
    y1, x1 = y1_ref[...], x1_ref[...]

    def pair(x, j, bit0):
        lo = pltpu.roll(x, (-j) % _NFP, 1)
        hi = pltpu.roll(x, j, 1)
        return jnp.where(bit0, lo, hi)

    k = 2
    while k <= _NFP:
        j = k // 2
        while j >= 1:
            bit0 = (ii & j) == 0
            keep_max = ((ii & k) == 0) == bit0
            ps = pair(s, j, bit0)
            pidx = pair(idx, j, bit0)
            py0 = pair(y0, j, bit0)
            px0 = pair(x0, j, bit0)
            py1 = pair(y1, j, bit0)
            px1 = pair(x1, j, bit0)
            greater = (s > ps) | ((s == ps) & (idx < pidx))
            take_self = keep_max == greater
            s = jnp.where(take_self, s, ps)
            idx = jnp.where(take_self, idx, pidx)
            y0 = jnp.where(take_self, y0, py0)
            x0 = jnp.where(take_self, x0, px0)
            y1 = jnp.where(take_self, y1, py1)
            x1 = jnp.where(take_self, x1, px1)
            j //= 2
        k *= 2

    sc_o[...] = s[:, :_M]
    cl_o[...] = (idx[:, :_M] // _M).astype(jnp.float32)
    y0_o[...] = y0[:, :_M]
    x0_o[...] = x0[:, :_M]
    y1_o[...] = y1[:, :_M]
    x1_o[...] = x1[:, :_M]


def _sort(sf, y0f, x0f, y1f, x1f):
    f32 = jnp.float32
    outs = [jax.ShapeDtypeStruct((_B, _M), f32) for _ in range(6)]
    return pl.pallas_call(
        _sort_body,
        out_shape=outs,
        compiler_params=pltpu.CompilerParams(
            vmem_limit_bytes=100 * 1024 * 1024),
    )(sf, y0f, x0f, y1f, x1f)


@jax.jit
def kernel(level_3, level_4, level_5):
    parts = []
    for x in (level_3, level_4, level_5):
        _, H, W, _ = x.shape
        parts.append(x.reshape(_B, H * W * 3, 85))
    d = jnp.concatenate(parts, axis=1)
    pad = _NP - _N
    bx = jnp.pad(d[..., 0], ((0, 0), (0, pad)))
    by = jnp.pad(d[..., 1], ((0, 0), (0, pad)))
    bw = jnp.pad(d[..., 2], ((0, 0), (0, pad)))
    bh = jnp.pad(d[..., 3], ((0, 0), (0, pad)))
    obj = jnp.pad(d[..., 4], ((0, 0), (0, pad)))
    cls_t = jnp.pad(jnp.transpose(d[..., 5:], (0, 2, 1)).reshape(_R, _N),
                    ((0, 0), (0, pad)))

    scores, t16, y0, x0, y1, x1 = _decode(cls_t, obj, bx, by, bw, bh)
    sc_c, y0c, x0c, y1c, x1c = _compact(scores, t16, y0, x0, y1, x1)
    ssv, y0s, x0s, y1s, x1s = _nms(sc_c, y0c, x0c, y1c, x1c)

    fpad = _NFP - _NF
    flat = lambda a, v: jnp.pad(a.reshape(_B, _NF), ((0, 0), (0, fpad)),
                                constant_values=v)
    sc, cl, fy0, fx0, fy1, fx1 = _sort(
        flat(ssv, -1.0), flat(y0s, 0.0), flat(x0s, 0.0),
        flat(y1s, 0.0), flat(x1s, 0.0))

    boxes = jnp.stack([fy0, fx0, fy1, fx1], axis=-1)
    return boxes, sc, cl
